# Initial kernel scaffold; baseline (speedup 1.0000x reference)
import jax
import jax.numpy as jnp
from jax.experimental import pallas as pl


def kernel(x, edge_index, eps1, W11, b11, W12, b12, eps2, W21, b21, W22, b22):
    raise NotImplementedError("write your pallas kernel here")


# speedup vs baseline: 3.2528x; 3.2528x over previous
import functools

import jax
import jax.numpy as jnp
from jax import lax
from jax.experimental import pallas as pl
from jax.experimental.pallas import tpu as pltpu
from jax.experimental.pallas import tpu_sc as plsc

_NC = 2
_NS = 16
_EDGE_CHUNK = 80


def _sc_aggregate(x_flat, srcoff, dst):
    twon, dh = x_flat.shape
    n = twon // _NC
    e = dst.shape[0]
    rows_per_tile = (n // _NS) & ~7
    rows_tail = n - _NS * rows_per_tile
    e_per_tile = e // _NS
    n_chunks = e_per_tile // _EDGE_CHUNK

    mesh = plsc.VectorSubcoreMesh(core_axis_name="c", subcore_axis_name="s")

    @functools.partial(
        pl.kernel,
        out_type=jax.ShapeDtypeStruct((twon, dh), jnp.float32),
        mesh=mesh,
        scratch_types=[
            pltpu.VMEM_SHARED((n, dh), jnp.float32),
            pltpu.VMEM((_EDGE_CHUNK,), jnp.int32),
            pltpu.VMEM((_EDGE_CHUNK,), jnp.int32),
            pltpu.VMEM((_EDGE_CHUNK, dh), jnp.float32),
            pltpu.SemaphoreType.DMA,
        ],
    )
    def agg_kernel(x_hbm, srcoff_hbm, dst_hbm, out_hbm, acc, src_v, dst_v,
                   rows_v, sem):
        c = lax.axis_index("c")
        s = lax.axis_index("s")
        row0 = s * rows_per_tile
        pltpu.sync_copy(x_hbm.at[pl.ds(c * n + row0, rows_per_tile)],
                        acc.at[pl.ds(row0, rows_per_tile)])
        if rows_tail:
            @pl.when(s == _NS - 1)
            def _():
                t0 = _NS * rows_per_tile
                pltpu.sync_copy(x_hbm.at[pl.ds(c * n + t0, rows_tail)],
                                acc.at[pl.ds(t0, rows_tail)])
        plsc.subcore_barrier()
        e0 = s * e_per_tile

        def body(i, carry):
            off = e0 + i * _EDGE_CHUNK
            pltpu.sync_copy(srcoff_hbm.at[pl.ds(c * e + off, _EDGE_CHUNK)],
                            src_v)
            pltpu.sync_copy(dst_hbm.at[pl.ds(off, _EDGE_CHUNK)], dst_v)
            pltpu.async_copy(x_hbm.at[src_v], rows_v, sem).wait()
            pltpu.sync_copy(rows_v, acc.at[dst_v], add=True)
            return carry

        lax.fori_loop(0, n_chunks, body, 0)
        plsc.subcore_barrier()
        pltpu.sync_copy(acc.at[pl.ds(row0, rows_per_tile)],
                        out_hbm.at[pl.ds(c * n + row0, rows_per_tile)])
        if rows_tail:
            @pl.when(s == _NS - 1)
            def _():
                t0 = _NS * rows_per_tile
                pltpu.sync_copy(acc.at[pl.ds(t0, rows_tail)],
                                out_hbm.at[pl.ds(c * n + t0, rows_tail)])

    return agg_kernel(x_flat, srcoff, dst)


def _tc_fold(w1, b1, w2, b2):
    d, h = w1.shape
    d2 = w2.shape[1]

    def fold_kernel(w1_ref, b1_ref, w2_ref, b2_ref, wc_ref, bc_ref):
        wc_ref[...] = jnp.dot(w1_ref[...], w2_ref[...],
                              preferred_element_type=jnp.float32)
        bc_ref[...] = jnp.dot(b1_ref[...], w2_ref[...],
                              preferred_element_type=jnp.float32) + b2_ref[...]

    return pl.pallas_call(
        fold_kernel,
        out_shape=(jax.ShapeDtypeStruct((d, d2), jnp.float32),
                   jax.ShapeDtypeStruct((1, d2), jnp.float32)),
    )(w1, b1.reshape(1, h), w2, b2.reshape(1, d2))


def _tc_dense(acc3, x3, eps, wc, bc, bn):
    _, n, dh = acc3.shape
    d = 2 * dh
    grid = (n // bn,)

    def dense_kernel(eps_ref, a0, a1, x0, x1, wc_ref, bc_ref, full_ref,
                     split_ref):
        ev = eps_ref[0]
        h0 = a0[0] + ev * x0[0]
        h1 = a1[0] + ev * x1[0]
        r = jnp.dot(h0, wc_ref[:dh, :], preferred_element_type=jnp.float32)
        r = r + jnp.dot(h1, wc_ref[dh:, :],
                        preferred_element_type=jnp.float32)
        r = jnp.maximum(r + bc_ref[...], 0.0)
        full_ref[...] = r
        split_ref[0] = r[:, :dh]
        split_ref[1] = r[:, dh:]

    return pl.pallas_call(
        dense_kernel,
        grid=grid,
        in_specs=[
            pl.BlockSpec(memory_space=pltpu.SMEM),
            pl.BlockSpec((1, bn, dh), lambda i: (0, i, 0)),
            pl.BlockSpec((1, bn, dh), lambda i: (1, i, 0)),
            pl.BlockSpec((1, bn, dh), lambda i: (0, i, 0)),
            pl.BlockSpec((1, bn, dh), lambda i: (1, i, 0)),
            pl.BlockSpec((d, d), lambda i: (0, 0)),
            pl.BlockSpec((1, d), lambda i: (0, 0)),
        ],
        out_specs=(
            pl.BlockSpec((bn, d), lambda i: (i, 0)),
            pl.BlockSpec((2, bn, dh), lambda i: (0, i, 0)),
        ),
        out_shape=(jax.ShapeDtypeStruct((n, d), jnp.float32),
                   jax.ShapeDtypeStruct((2, n, dh), jnp.float32)),
    )(eps.reshape(1), acc3, acc3, x3, x3, wc, bc)


def kernel(x, edge_index, eps1, W11, b11, W12, b12, eps2, W21, b21, W22,
           b22):
    n, d = x.shape
    dh = d // 2
    src = edge_index[0]
    dst = edge_index[1]
    srcoff = jnp.concatenate([src, src + n])

    wc1, bc1 = _tc_fold(W11, b11, W12, b12)
    wc2, bc2 = _tc_fold(W21, b21, W22, b22)

    x_split = jnp.concatenate([x[:, :dh], x[:, dh:]], axis=0)

    bn = 1000
    acc1 = _sc_aggregate(x_split, srcoff, dst)
    x1_full, x1_split = _tc_dense(acc1.reshape(_NC, n, dh),
                                  x_split.reshape(_NC, n, dh),
                                  eps1, wc1, bc1, bn)
    acc2 = _sc_aggregate(x1_split.reshape(_NC * n, dh), srcoff, dst)
    x2_full, _ = _tc_dense(acc2.reshape(_NC, n, dh), x1_split, eps2, wc2,
                           bc2, bn)

    return jnp.concatenate([x1_full[:, :, None], x2_full[:, :, None]],
                           axis=2)


# speedup vs baseline: 6.7924x; 2.0882x over previous
import functools

import jax
import jax.numpy as jnp
from jax import lax
from jax.experimental import pallas as pl
from jax.experimental.pallas import tpu as pltpu
from jax.experimental.pallas import tpu_sc as plsc

_NC = 2
_NS = 16
_K = 128
_JUNK = 16


def _sc_aggregate(x_flat, srcoff, dstp, chunks):
    twon, dh = x_flat.shape
    n = twon // _NC
    rows_per_tile = (n // _NS) & ~7
    rows_tail = n - _NS * rows_per_tile
    nsteps = chunks // 2

    mesh = plsc.VectorSubcoreMesh(core_axis_name="c", subcore_axis_name="s")

    @functools.partial(
        pl.kernel,
        out_type=jax.ShapeDtypeStruct((twon, dh), jnp.float32),
        mesh=mesh,
        scratch_types=[
            pltpu.VMEM_SHARED((n + _JUNK, dh), jnp.float32),
            pltpu.VMEM((_K,), jnp.int32),
            pltpu.VMEM((_K,), jnp.int32),
            pltpu.VMEM((_K,), jnp.int32),
            pltpu.VMEM((_K,), jnp.int32),
            pltpu.VMEM((_K, dh), jnp.float32),
            pltpu.VMEM((_K, dh), jnp.float32),
            pltpu.SemaphoreType.DMA,
            pltpu.SemaphoreType.DMA,
            pltpu.SemaphoreType.DMA,
            pltpu.SemaphoreType.DMA,
            pltpu.SemaphoreType.DMA,
            pltpu.SemaphoreType.DMA,
        ],
    )
    def agg_kernel(x_hbm, srcoff_hbm, dst_hbm, out_hbm, acc, si0, si1,
                   di0, di1, g0, g1, is0, is1, ds0, ds1, gs0, gs1):
        c = lax.axis_index("c")
        s = lax.axis_index("s")
        row0 = s * rows_per_tile
        pltpu.async_copy(srcoff_hbm.at[c, s, 0], si0, is0)
        pltpu.async_copy(dst_hbm.at[s, 0], di0, ds0)
        pltpu.async_copy(srcoff_hbm.at[c, s, 1], si1, is1)
        pltpu.async_copy(dst_hbm.at[s, 1], di1, ds1)
        pltpu.sync_copy(x_hbm.at[pl.ds(c * n + row0, rows_per_tile)],
                        acc.at[pl.ds(row0, rows_per_tile)])
        if rows_tail:
            @pl.when(s == _NS - 1)
            def _():
                t0 = _NS * rows_per_tile
                pltpu.sync_copy(x_hbm.at[pl.ds(c * n + t0, rows_tail)],
                                acc.at[pl.ds(t0, rows_tail)])
        plsc.subcore_barrier()

        pltpu.make_async_copy(srcoff_hbm.at[c, s, 0], si0, is0).wait()
        pltpu.async_copy(x_hbm.at[si0], g0, gs0)
        pltpu.make_async_copy(srcoff_hbm.at[c, s, 1], si1, is1).wait()
        pltpu.async_copy(x_hbm.at[si1], g1, gs1)

        def body(j, carry):
            j0 = 2 * j
            pltpu.make_async_copy(x_hbm.at[si0], g0, gs0).wait()

            @pl.when(j0 + 2 < chunks)
            def _():
                pltpu.async_copy(srcoff_hbm.at[c, s, j0 + 2], si0, is0)

            pltpu.make_async_copy(dst_hbm.at[s, j0], di0, ds0).wait()
            pltpu.sync_copy(g0, acc.at[di0], add=True)

            @pl.when(j0 + 2 < chunks)
            def _():
                pltpu.async_copy(dst_hbm.at[s, j0 + 2], di0, ds0)
                pltpu.make_async_copy(srcoff_hbm.at[c, s, j0 + 2], si0,
                                      is0).wait()
                pltpu.async_copy(x_hbm.at[si0], g0, gs0)

            pltpu.make_async_copy(x_hbm.at[si1], g1, gs1).wait()

            @pl.when(j0 + 3 < chunks)
            def _():
                pltpu.async_copy(srcoff_hbm.at[c, s, j0 + 3], si1, is1)

            pltpu.make_async_copy(dst_hbm.at[s, j0 + 1], di1, ds1).wait()
            pltpu.sync_copy(g1, acc.at[di1], add=True)

            @pl.when(j0 + 3 < chunks)
            def _():
                pltpu.async_copy(dst_hbm.at[s, j0 + 3], di1, ds1)
                pltpu.make_async_copy(srcoff_hbm.at[c, s, j0 + 3], si1,
                                      is1).wait()
                pltpu.async_copy(x_hbm.at[si1], g1, gs1)

            return carry

        lax.fori_loop(0, nsteps, body, 0)
        plsc.subcore_barrier()
        pltpu.sync_copy(acc.at[pl.ds(row0, rows_per_tile)],
                        out_hbm.at[pl.ds(c * n + row0, rows_per_tile)])
        if rows_tail:
            @pl.when(s == _NS - 1)
            def _():
                t0 = _NS * rows_per_tile
                pltpu.sync_copy(acc.at[pl.ds(t0, rows_tail)],
                                out_hbm.at[pl.ds(c * n + t0, rows_tail)])

    return agg_kernel(x_flat, srcoff, dstp)


def _tc_fold(w1, b1, w2, b2):
    d, h = w1.shape
    d2 = w2.shape[1]

    def fold_kernel(w1_ref, b1_ref, w2_ref, b2_ref, wc_ref, bc_ref):
        wc_ref[...] = jnp.dot(w1_ref[...], w2_ref[...],
                              preferred_element_type=jnp.float32)
        bc_ref[...] = jnp.dot(b1_ref[...], w2_ref[...],
                              preferred_element_type=jnp.float32) + b2_ref[...]

    return pl.pallas_call(
        fold_kernel,
        out_shape=(jax.ShapeDtypeStruct((d, d2), jnp.float32),
                   jax.ShapeDtypeStruct((1, d2), jnp.float32)),
    )(w1, b1.reshape(1, h), w2, b2.reshape(1, d2))


def _tc_dense(acc3, x3, eps, wc, bc, bn):
    _, n, dh = acc3.shape
    d = 2 * dh
    grid = (n // bn,)

    def dense_kernel(eps_ref, a0, a1, x0, x1, wc_ref, bc_ref, full_ref,
                     split_ref):
        ev = eps_ref[0]
        h0 = a0[0] + ev * x0[0]
        h1 = a1[0] + ev * x1[0]
        r = jnp.dot(h0, wc_ref[:dh, :], preferred_element_type=jnp.float32)
        r = r + jnp.dot(h1, wc_ref[dh:, :],
                        preferred_element_type=jnp.float32)
        r = jnp.maximum(r + bc_ref[...], 0.0)
        full_ref[...] = r
        split_ref[0] = r[:, :dh]
        split_ref[1] = r[:, dh:]

    return pl.pallas_call(
        dense_kernel,
        grid=grid,
        in_specs=[
            pl.BlockSpec(memory_space=pltpu.SMEM),
            pl.BlockSpec((1, bn, dh), lambda i: (0, i, 0)),
            pl.BlockSpec((1, bn, dh), lambda i: (1, i, 0)),
            pl.BlockSpec((1, bn, dh), lambda i: (0, i, 0)),
            pl.BlockSpec((1, bn, dh), lambda i: (1, i, 0)),
            pl.BlockSpec((d, d), lambda i: (0, 0)),
            pl.BlockSpec((1, d), lambda i: (0, 0)),
        ],
        out_specs=(
            pl.BlockSpec((bn, d), lambda i: (i, 0)),
            pl.BlockSpec((2, bn, dh), lambda i: (0, i, 0)),
        ),
        out_shape=(jax.ShapeDtypeStruct((n, d), jnp.float32),
                   jax.ShapeDtypeStruct((2, n, dh), jnp.float32)),
    )(eps.reshape(1), acc3, acc3, x3, x3, wc, bc)


def kernel(x, edge_index, eps1, W11, b11, W12, b12, eps2, W21, b21, W22,
           b22):
    n, d = x.shape
    dh = d // 2
    e = edge_index.shape[1]
    src = edge_index[0]
    dst = edge_index[1]

    e_per_tile = e // _NS
    chunks = -(-e_per_tile // _K)
    chunks += chunks % 2
    pad = chunks * _K - e_per_tile
    src2 = src.reshape(_NS, e_per_tile)
    dst2 = dst.reshape(_NS, e_per_tile)
    if pad:
        pad_src = jnp.broadcast_to((jnp.arange(pad, dtype=jnp.int32) * 64)
                                   % n, (_NS, pad))
        pad_dst = jnp.broadcast_to(
            n + jnp.arange(pad, dtype=jnp.int32) % _JUNK, (_NS, pad))
        src2 = jnp.concatenate([src2, pad_src], axis=1)
        dst2 = jnp.concatenate([dst2, pad_dst], axis=1)
    srcoff = jnp.stack([src2, src2 + n]).reshape(2, _NS, chunks, _K)
    dstp = dst2.reshape(_NS, chunks, _K)

    wc1, bc1 = _tc_fold(W11, b11, W12, b12)
    wc2, bc2 = _tc_fold(W21, b21, W22, b22)

    x_split = jnp.concatenate([x[:, :dh], x[:, dh:]], axis=0)

    bn = 1000
    acc1 = _sc_aggregate(x_split, srcoff, dstp, chunks)
    x1_full, x1_split = _tc_dense(acc1.reshape(_NC, n, dh),
                                  x_split.reshape(_NC, n, dh),
                                  eps1, wc1, bc1, bn)
    acc2 = _sc_aggregate(x1_split.reshape(_NC * n, dh), srcoff, dstp,
                         chunks)
    x2_full, _ = _tc_dense(acc2.reshape(_NC, n, dh), x1_split, eps2, wc2,
                           bc2, bn)

    return jnp.concatenate([x1_full[:, :, None], x2_full[:, :, None]],
                           axis=2)


# speedup vs baseline: 6.7986x; 1.0009x over previous
import functools

import jax
import jax.numpy as jnp
from jax import lax
from jax.experimental import pallas as pl
from jax.experimental.pallas import tpu as pltpu
from jax.experimental.pallas import tpu_sc as plsc

_NC = 2
_NS = 16
_K = 128
_JUNK = 16


def _sc_aggregate(x_flat, srcoff, dstp, chunks):
    twon, dh = x_flat.shape
    n = twon // _NC
    rows_per_tile = (n // _NS) & ~7
    rows_tail = n - _NS * rows_per_tile
    nsteps = chunks // 2

    mesh = plsc.VectorSubcoreMesh(core_axis_name="c", subcore_axis_name="s")

    @functools.partial(
        pl.kernel,
        out_type=jax.ShapeDtypeStruct((twon, dh), jnp.float32),
        mesh=mesh,
        scratch_types=[
            pltpu.VMEM_SHARED((n + _JUNK, dh), jnp.float32),
            pltpu.VMEM((_K,), jnp.int32),
            pltpu.VMEM((_K,), jnp.int32),
            pltpu.VMEM((_K,), jnp.int32),
            pltpu.VMEM((_K,), jnp.int32),
            pltpu.VMEM((_K, dh), jnp.float32),
            pltpu.VMEM((_K, dh), jnp.float32),
            pltpu.SemaphoreType.DMA,
            pltpu.SemaphoreType.DMA,
            pltpu.SemaphoreType.DMA,
            pltpu.SemaphoreType.DMA,
            pltpu.SemaphoreType.DMA,
            pltpu.SemaphoreType.DMA,
        ],
    )
    def agg_kernel(x_hbm, srcoff_hbm, dst_hbm, out_hbm, acc, si0, si1,
                   di0, di1, g0, g1, is0, is1, ds0, ds1, gs0, gs1):
        c = lax.axis_index("c")
        s = lax.axis_index("s")
        row0 = s * rows_per_tile
        pltpu.async_copy(srcoff_hbm.at[c, s, 0], si0, is0)
        pltpu.async_copy(dst_hbm.at[s, 0], di0, ds0)
        pltpu.async_copy(srcoff_hbm.at[c, s, 1], si1, is1)
        pltpu.async_copy(dst_hbm.at[s, 1], di1, ds1)
        pltpu.sync_copy(x_hbm.at[pl.ds(c * n + row0, rows_per_tile)],
                        acc.at[pl.ds(row0, rows_per_tile)])
        if rows_tail:
            @pl.when(s == _NS - 1)
            def _():
                t0 = _NS * rows_per_tile
                pltpu.sync_copy(x_hbm.at[pl.ds(c * n + t0, rows_tail)],
                                acc.at[pl.ds(t0, rows_tail)])
        plsc.subcore_barrier()

        pltpu.make_async_copy(srcoff_hbm.at[c, s, 0], si0, is0).wait()
        pltpu.async_copy(x_hbm.at[si0], g0, gs0)
        pltpu.make_async_copy(srcoff_hbm.at[c, s, 1], si1, is1).wait()
        pltpu.async_copy(x_hbm.at[si1], g1, gs1)

        def body(j, carry):
            j0 = 2 * j
            pltpu.make_async_copy(x_hbm.at[si0], g0, gs0).wait()

            @pl.when(j0 + 2 < chunks)
            def _():
                pltpu.async_copy(srcoff_hbm.at[c, s, j0 + 2], si0, is0)

            pltpu.make_async_copy(dst_hbm.at[s, j0], di0, ds0).wait()
            pltpu.sync_copy(g0, acc.at[di0], add=True)

            @pl.when(j0 + 2 < chunks)
            def _():
                pltpu.async_copy(dst_hbm.at[s, j0 + 2], di0, ds0)
                pltpu.make_async_copy(srcoff_hbm.at[c, s, j0 + 2], si0,
                                      is0).wait()
                pltpu.async_copy(x_hbm.at[si0], g0, gs0)

            pltpu.make_async_copy(x_hbm.at[si1], g1, gs1).wait()

            @pl.when(j0 + 3 < chunks)
            def _():
                pltpu.async_copy(srcoff_hbm.at[c, s, j0 + 3], si1, is1)

            pltpu.make_async_copy(dst_hbm.at[s, j0 + 1], di1, ds1).wait()
            pltpu.sync_copy(g1, acc.at[di1], add=True)

            @pl.when(j0 + 3 < chunks)
            def _():
                pltpu.async_copy(dst_hbm.at[s, j0 + 3], di1, ds1)
                pltpu.make_async_copy(srcoff_hbm.at[c, s, j0 + 3], si1,
                                      is1).wait()
                pltpu.async_copy(x_hbm.at[si1], g1, gs1)

            return carry

        lax.fori_loop(0, nsteps, body, 0)
        plsc.subcore_barrier()
        pltpu.sync_copy(acc.at[pl.ds(row0, rows_per_tile)],
                        out_hbm.at[pl.ds(c * n + row0, rows_per_tile)])
        if rows_tail:
            @pl.when(s == _NS - 1)
            def _():
                t0 = _NS * rows_per_tile
                pltpu.sync_copy(acc.at[pl.ds(t0, rows_tail)],
                                out_hbm.at[pl.ds(c * n + t0, rows_tail)])

    return agg_kernel(x_flat, srcoff, dstp)


def _tc_fold(w1, b1, w2, b2):
    d, h = w1.shape
    d2 = w2.shape[1]

    def fold_kernel(w1_ref, b1_ref, w2_ref, b2_ref, wc_ref, bc_ref):
        wc_ref[...] = jnp.dot(w1_ref[...], w2_ref[...],
                              preferred_element_type=jnp.float32)
        bc_ref[...] = jnp.dot(b1_ref[...], w2_ref[...],
                              preferred_element_type=jnp.float32) + b2_ref[...]

    return pl.pallas_call(
        fold_kernel,
        out_shape=(jax.ShapeDtypeStruct((d, d2), jnp.float32),
                   jax.ShapeDtypeStruct((1, d2), jnp.float32)),
    )(w1, b1.reshape(1, h), w2, b2.reshape(1, d2))


def _tc_dense(acc3, x3, eps, wc, bc, bn):
    _, n, dh = acc3.shape
    d = 2 * dh
    grid = (n // bn,)

    def dense_kernel(eps_ref, a0, a1, x0, x1, wc_ref, bc_ref, full_ref,
                     split_ref):
        ev = eps_ref[0]
        h0 = a0[0] + ev * x0[0]
        h1 = a1[0] + ev * x1[0]
        r = jnp.dot(h0, wc_ref[:dh, :], preferred_element_type=jnp.float32)
        r = r + jnp.dot(h1, wc_ref[dh:, :],
                        preferred_element_type=jnp.float32)
        r = jnp.maximum(r + bc_ref[...], 0.0)
        full_ref[...] = r
        split_ref[0] = r[:, :dh]
        split_ref[1] = r[:, dh:]

    return pl.pallas_call(
        dense_kernel,
        grid=grid,
        in_specs=[
            pl.BlockSpec(memory_space=pltpu.SMEM),
            pl.BlockSpec((1, bn, dh), lambda i: (0, i, 0)),
            pl.BlockSpec((1, bn, dh), lambda i: (1, i, 0)),
            pl.BlockSpec((1, bn, dh), lambda i: (0, i, 0)),
            pl.BlockSpec((1, bn, dh), lambda i: (1, i, 0)),
            pl.BlockSpec((d, d), lambda i: (0, 0)),
            pl.BlockSpec((1, d), lambda i: (0, 0)),
        ],
        out_specs=(
            pl.BlockSpec((bn, d), lambda i: (i, 0)),
            pl.BlockSpec((2, bn, dh), lambda i: (0, i, 0)),
        ),
        out_shape=(jax.ShapeDtypeStruct((n, d), jnp.float32),
                   jax.ShapeDtypeStruct((2, n, dh), jnp.float32)),
    )(eps.reshape(1), acc3, acc3, x3, x3, wc, bc)


def _tc_split(x, bn):
    n, d = x.shape
    dh = d // 2

    def split_kernel(x_ref, out_ref):
        out_ref[0] = x_ref[:, :dh]
        out_ref[1] = x_ref[:, dh:]

    return pl.pallas_call(
        split_kernel,
        grid=(n // bn,),
        in_specs=[pl.BlockSpec((bn, d), lambda i: (i, 0))],
        out_specs=pl.BlockSpec((2, bn, dh), lambda i: (0, i, 0)),
        out_shape=jax.ShapeDtypeStruct((2, n, dh), jnp.float32),
    )(x)


def _tc_dense_final(acc3, x3, eps, wc, bc, x1_full, bn):
    _, n, dh = acc3.shape
    d = 2 * dh
    grid = (n // bn,)

    def dense_kernel(eps_ref, a0, a1, x0, x1, wc_ref, bc_ref, x1f_ref,
                     inter_ref):
        ev = eps_ref[0]
        h0 = a0[0] + ev * x0[0]
        h1 = a1[0] + ev * x1[0]
        r = jnp.dot(h0, wc_ref[:dh, :], preferred_element_type=jnp.float32)
        r = r + jnp.dot(h1, wc_ref[dh:, :],
                        preferred_element_type=jnp.float32)
        r = jnp.maximum(r + bc_ref[...], 0.0)
        h2 = jnp.concatenate([x1f_ref[...], r], axis=1)
        rowi = lax.broadcasted_iota(jnp.int32, (2 * d, 2 * d), 0)
        coli = lax.broadcasted_iota(jnp.int32, (2 * d, 2 * d), 1)
        target = jnp.where(rowi < d, 2 * rowi, 2 * (rowi - d) + 1)
        perm = (coli == target).astype(jnp.float32)
        inter_ref[...] = jnp.dot(h2, perm,
                                 preferred_element_type=jnp.float32)

    return pl.pallas_call(
        dense_kernel,
        grid=grid,
        in_specs=[
            pl.BlockSpec(memory_space=pltpu.SMEM),
            pl.BlockSpec((1, bn, dh), lambda i: (0, i, 0)),
            pl.BlockSpec((1, bn, dh), lambda i: (1, i, 0)),
            pl.BlockSpec((1, bn, dh), lambda i: (0, i, 0)),
            pl.BlockSpec((1, bn, dh), lambda i: (1, i, 0)),
            pl.BlockSpec((d, d), lambda i: (0, 0)),
            pl.BlockSpec((1, d), lambda i: (0, 0)),
            pl.BlockSpec((bn, d), lambda i: (i, 0)),
        ],
        out_specs=pl.BlockSpec((bn, 2 * d), lambda i: (i, 0)),
        out_shape=jax.ShapeDtypeStruct((n, 2 * d), jnp.float32),
    )(eps.reshape(1), acc3, acc3, x3, x3, wc, bc, x1_full)


def kernel(x, edge_index, eps1, W11, b11, W12, b12, eps2, W21, b21, W22,
           b22):
    n, d = x.shape
    dh = d // 2
    e = edge_index.shape[1]
    src = edge_index[0]
    dst = edge_index[1]

    e_per_tile = e // _NS
    chunks = -(-e_per_tile // _K)
    chunks += chunks % 2
    pad = chunks * _K - e_per_tile
    src2 = src.reshape(_NS, e_per_tile)
    dst2 = dst.reshape(_NS, e_per_tile)
    if pad:
        pad_src = jnp.broadcast_to((jnp.arange(pad, dtype=jnp.int32) * 64)
                                   % n, (_NS, pad))
        pad_dst = jnp.broadcast_to(
            n + jnp.arange(pad, dtype=jnp.int32) % _JUNK, (_NS, pad))
        src2 = jnp.concatenate([src2, pad_src], axis=1)
        dst2 = jnp.concatenate([dst2, pad_dst], axis=1)
    srcoff = jnp.stack([src2, src2 + n]).reshape(2, _NS, chunks, _K)
    dstp = dst2.reshape(_NS, chunks, _K)

    wc1, bc1 = _tc_fold(W11, b11, W12, b12)
    wc2, bc2 = _tc_fold(W21, b21, W22, b22)

    x_split = jnp.concatenate([x[:, :dh], x[:, dh:]], axis=0)

    bn = 1000
    acc1 = _sc_aggregate(x_split, srcoff, dstp, chunks)
    x1_full, x1_split = _tc_dense(acc1.reshape(_NC, n, dh),
                                  x_split.reshape(_NC, n, dh),
                                  eps1, wc1, bc1, bn)
    acc2 = _sc_aggregate(x1_split.reshape(_NC * n, dh), srcoff, dstp,
                         chunks)
    x2_full, _ = _tc_dense(acc2.reshape(_NC, n, dh), x1_split, eps2, wc2,
                           bc2, bn)

    return jnp.concatenate([x1_full[:, :, None], x2_full[:, :, None]],
                           axis=2)
